# Initial kernel scaffold; baseline (speedup 1.0000x reference)
import jax
import jax.numpy as jnp
from jax.experimental import pallas as pl


def kernel(hidden_states, ln_weight, ln_bias, gate_weight):
    raise NotImplementedError("write your pallas kernel here")


# speedup vs baseline: 2.3556x; 2.3556x over previous
import functools

import jax
import jax.numpy as jnp
from jax import lax
from jax.experimental import pallas as pl
from jax.experimental.pallas import tpu as pltpu

EPS_ = 1e-4
BLK = 512


def _tc_router_kernel(x_ref, w_ref, b_ref, gt_ref, logits_ref, probs_ref, idx_ref):
    x = x_ref[...]
    mean = jnp.mean(x, axis=1, keepdims=True)
    xc = x - mean
    var = jnp.mean(xc * xc, axis=1, keepdims=True)
    hn = xc / jnp.sqrt(var + 1e-5) * w_ref[...] + b_ref[...]
    hn = jnp.clip(hn, -50.0, 50.0)
    logits = jax.lax.dot_general(
        hn, gt_ref[...], (((1,), (0,)), ((), ())),
        preferred_element_type=jnp.float32,
    )
    logits = jnp.clip(logits, -10.0, 10.0)
    logits_ref[...] = logits

    m = jnp.max(logits, axis=1, keepdims=True)
    e = jnp.exp(logits - m)
    z = jnp.sum(e, axis=1, keepdims=True)
    p = jnp.clip(e / z, EPS_, 1.0)
    ncols = logits.shape[1]
    iota = lax.broadcasted_iota(jnp.int32, p.shape, 1)
    m1 = jnp.max(p, axis=1, keepdims=True)
    i1 = jnp.min(jnp.where(p == m1, iota, ncols), axis=1, keepdims=True)
    pm = jnp.where(iota == i1, -1.0, p)
    m2 = jnp.max(pm, axis=1, keepdims=True)
    i2 = jnp.min(jnp.where(pm == m2, iota, ncols), axis=1, keepdims=True)
    s = jnp.maximum(m1 + m2, EPS_)
    probs_ref[...] = jnp.concatenate([m1 / s, m2 / s], axis=1)
    idx_ref[...] = jnp.concatenate([i1, i2], axis=1)


@jax.jit
def kernel(hidden_states, ln_weight, ln_bias, gate_weight):
    B, S, D = hidden_states.shape
    N = B * S
    x = hidden_states.reshape(N, D)
    w = ln_weight.reshape(1, D)
    b = ln_bias.reshape(1, D)
    gt = gate_weight.T
    E = gate_weight.shape[0]
    grid = (N // BLK,)
    logits, probs, idx = pl.pallas_call(
        _tc_router_kernel,
        grid=grid,
        in_specs=[
            pl.BlockSpec((BLK, D), lambda i: (i, 0)),
            pl.BlockSpec((1, D), lambda i: (0, 0)),
            pl.BlockSpec((1, D), lambda i: (0, 0)),
            pl.BlockSpec((D, E), lambda i: (0, 0)),
        ],
        out_specs=[
            pl.BlockSpec((BLK, E), lambda i: (i, 0)),
            pl.BlockSpec((BLK, 2), lambda i: (i, 0)),
            pl.BlockSpec((BLK, 2), lambda i: (i, 0)),
        ],
        out_shape=[
            jax.ShapeDtypeStruct((N, E), jnp.float32),
            jax.ShapeDtypeStruct((N, 2), jnp.float32),
            jax.ShapeDtypeStruct((N, 2), jnp.int32),
        ],
    )(x, w, b, gt)
    return probs, idx, logits


# speedup vs baseline: 2.3677x; 1.0051x over previous
import functools

import jax
import jax.numpy as jnp
from jax import lax
from jax.experimental import pallas as pl
from jax.experimental.pallas import tpu as pltpu

EPS_ = 1e-4
BLK = 512


def _tc_router_kernel(x_ref, w_ref, b_ref, gt_ref, logits_ref, probs_ref, idx_ref):
    x = x_ref[...]
    mean = jnp.mean(x, axis=1, keepdims=True)
    xc = x - mean
    var = jnp.mean(xc * xc, axis=1, keepdims=True)
    hn = xc * lax.rsqrt(var + 1e-5) * w_ref[...] + b_ref[...]
    hn = jnp.clip(hn, -50.0, 50.0)
    logits = jax.lax.dot_general(
        hn, gt_ref[...], (((1,), (0,)), ((), ())),
        preferred_element_type=jnp.float32,
    )
    logits = jnp.clip(logits, -10.0, 10.0)
    logits_ref[...] = logits

    m = jnp.max(logits, axis=1, keepdims=True)
    e = jnp.exp(logits - m)
    z = jnp.sum(e, axis=1, keepdims=True)
    p = jnp.clip(e / z, EPS_, 1.0)
    ncols = logits.shape[1]
    iota = lax.broadcasted_iota(jnp.int32, p.shape, 1)
    m1 = jnp.max(p, axis=1, keepdims=True)
    i1 = jnp.min(jnp.where(p == m1, iota, ncols), axis=1, keepdims=True)
    pm = jnp.where(iota == i1, -1.0, p)
    m2 = jnp.max(pm, axis=1, keepdims=True)
    i2 = jnp.min(jnp.where(pm == m2, iota, ncols), axis=1, keepdims=True)
    s = jnp.maximum(m1 + m2, EPS_)
    probs_ref[...] = jnp.concatenate([m1 / s, m2 / s], axis=1)
    idx_ref[...] = jnp.concatenate([i1, i2], axis=1)


@jax.jit
def kernel(hidden_states, ln_weight, ln_bias, gate_weight):
    B, S, D = hidden_states.shape
    N = B * S
    x = hidden_states.reshape(N, D)
    w = ln_weight.reshape(1, D)
    b = ln_bias.reshape(1, D)
    gt = gate_weight.T
    E = gate_weight.shape[0]
    grid = (N // BLK,)
    logits, probs, idx = pl.pallas_call(
        _tc_router_kernel,
        grid=grid,
        in_specs=[
            pl.BlockSpec((BLK, D), lambda i: (i, 0)),
            pl.BlockSpec((1, D), lambda i: (0, 0)),
            pl.BlockSpec((1, D), lambda i: (0, 0)),
            pl.BlockSpec((D, E), lambda i: (0, 0)),
        ],
        out_specs=[
            pl.BlockSpec((BLK, E), lambda i: (i, 0)),
            pl.BlockSpec((BLK, 2), lambda i: (i, 0)),
            pl.BlockSpec((BLK, 2), lambda i: (i, 0)),
        ],
        out_shape=[
            jax.ShapeDtypeStruct((N, E), jnp.float32),
            jax.ShapeDtypeStruct((N, 2), jnp.float32),
            jax.ShapeDtypeStruct((N, 2), jnp.int32),
        ],
    )(x, w, b, gt)
    return probs, idx, logits


# speedup vs baseline: 2.6875x; 1.1350x over previous
import functools

import jax
import jax.numpy as jnp
from jax import lax
from jax.experimental import pallas as pl
from jax.experimental.pallas import tpu as pltpu

EPS_ = 1e-4
BLK = 1024


def _tc_router_kernel(x_ref, w_ref, b_ref, gt_ref, logits_ref, probs_ref, idx_ref):
    x = x_ref[...]
    mean = jnp.mean(x, axis=1, keepdims=True)
    xc = x - mean
    var = jnp.mean(xc * xc, axis=1, keepdims=True)
    hn = xc * lax.rsqrt(var + 1e-5) * w_ref[...] + b_ref[...]
    hn = jnp.clip(hn, -50.0, 50.0)
    logits = jax.lax.dot_general(
        hn, gt_ref[...], (((1,), (0,)), ((), ())),
        preferred_element_type=jnp.float32,
    )
    logits = jnp.clip(logits, -10.0, 10.0)
    logits_ref[...] = logits

    m = jnp.max(logits, axis=1, keepdims=True)
    e = jnp.exp(logits - m)
    z = jnp.sum(e, axis=1, keepdims=True)
    p = jnp.clip(e / z, EPS_, 1.0)
    ncols = logits.shape[1]
    iota = lax.broadcasted_iota(jnp.int32, p.shape, 1)
    m1 = jnp.max(p, axis=1, keepdims=True)
    i1 = jnp.min(jnp.where(p == m1, iota, ncols), axis=1, keepdims=True)
    pm = jnp.where(iota == i1, -1.0, p)
    m2 = jnp.max(pm, axis=1, keepdims=True)
    i2 = jnp.min(jnp.where(pm == m2, iota, ncols), axis=1, keepdims=True)
    s = jnp.maximum(m1 + m2, EPS_)
    probs_ref[...] = jnp.concatenate([m1 / s, m2 / s], axis=1)
    idx_ref[...] = jnp.concatenate([i1, i2], axis=1)


@jax.jit
def kernel(hidden_states, ln_weight, ln_bias, gate_weight):
    B, S, D = hidden_states.shape
    N = B * S
    x = hidden_states.reshape(N, D)
    w = ln_weight.reshape(1, D)
    b = ln_bias.reshape(1, D)
    gt = gate_weight.T
    E = gate_weight.shape[0]
    grid = (N // BLK,)
    logits, probs, idx = pl.pallas_call(
        _tc_router_kernel,
        grid=grid,
        in_specs=[
            pl.BlockSpec((BLK, D), lambda i: (i, 0)),
            pl.BlockSpec((1, D), lambda i: (0, 0)),
            pl.BlockSpec((1, D), lambda i: (0, 0)),
            pl.BlockSpec((D, E), lambda i: (0, 0)),
        ],
        out_specs=[
            pl.BlockSpec((BLK, E), lambda i: (i, 0)),
            pl.BlockSpec((BLK, 2), lambda i: (i, 0)),
            pl.BlockSpec((BLK, 2), lambda i: (i, 0)),
        ],
        out_shape=[
            jax.ShapeDtypeStruct((N, E), jnp.float32),
            jax.ShapeDtypeStruct((N, 2), jnp.float32),
            jax.ShapeDtypeStruct((N, 2), jnp.int32),
        ],
    )(x, w, b, gt)
    return probs, idx, logits


# speedup vs baseline: 2.7085x; 1.0078x over previous
import functools

import jax
import jax.numpy as jnp
from jax import lax
from jax.experimental import pallas as pl
from jax.experimental.pallas import tpu as pltpu

EPS_ = 1e-4
BLK = 2048


def _tc_router_kernel(x_ref, w_ref, b_ref, gt_ref, logits_ref, probs_ref, idx_ref):
    x = x_ref[...]
    mean = jnp.mean(x, axis=1, keepdims=True)
    xc = x - mean
    var = jnp.mean(xc * xc, axis=1, keepdims=True)
    hn = xc * lax.rsqrt(var + 1e-5) * w_ref[...] + b_ref[...]
    hn = jnp.clip(hn, -50.0, 50.0)
    logits = jax.lax.dot_general(
        hn, gt_ref[...], (((1,), (0,)), ((), ())),
        preferred_element_type=jnp.float32,
    )
    logits = jnp.clip(logits, -10.0, 10.0)
    logits_ref[...] = logits

    m = jnp.max(logits, axis=1, keepdims=True)
    e = jnp.exp(logits - m)
    z = jnp.sum(e, axis=1, keepdims=True)
    p = jnp.clip(e / z, EPS_, 1.0)
    ncols = logits.shape[1]
    iota = lax.broadcasted_iota(jnp.int32, p.shape, 1)
    m1 = jnp.max(p, axis=1, keepdims=True)
    i1 = jnp.min(jnp.where(p == m1, iota, ncols), axis=1, keepdims=True)
    pm = jnp.where(iota == i1, -1.0, p)
    m2 = jnp.max(pm, axis=1, keepdims=True)
    i2 = jnp.min(jnp.where(pm == m2, iota, ncols), axis=1, keepdims=True)
    s = jnp.maximum(m1 + m2, EPS_)
    probs_ref[...] = jnp.concatenate([m1 / s, m2 / s], axis=1)
    idx_ref[...] = jnp.concatenate([i1, i2], axis=1)


@jax.jit
def kernel(hidden_states, ln_weight, ln_bias, gate_weight):
    B, S, D = hidden_states.shape
    N = B * S
    x = hidden_states.reshape(N, D)
    w = ln_weight.reshape(1, D)
    b = ln_bias.reshape(1, D)
    gt = gate_weight.T
    E = gate_weight.shape[0]
    grid = (N // BLK,)
    logits, probs, idx = pl.pallas_call(
        _tc_router_kernel,
        grid=grid,
        in_specs=[
            pl.BlockSpec((BLK, D), lambda i: (i, 0)),
            pl.BlockSpec((1, D), lambda i: (0, 0)),
            pl.BlockSpec((1, D), lambda i: (0, 0)),
            pl.BlockSpec((D, E), lambda i: (0, 0)),
        ],
        out_specs=[
            pl.BlockSpec((BLK, E), lambda i: (i, 0)),
            pl.BlockSpec((BLK, 2), lambda i: (i, 0)),
            pl.BlockSpec((BLK, 2), lambda i: (i, 0)),
        ],
        out_shape=[
            jax.ShapeDtypeStruct((N, E), jnp.float32),
            jax.ShapeDtypeStruct((N, 2), jnp.float32),
            jax.ShapeDtypeStruct((N, 2), jnp.int32),
        ],
    )(x, w, b, gt)
    return probs, idx, logits
